# speedup vs baseline: 1.2290x; 1.2290x over previous
import jax
import jax.numpy as jnp
from jax import lax
from jax.experimental import pallas as pl
from jax.experimental.pallas import tpu as pltpu
from jax.experimental.pallas import tpu_sc as plsc

_EMBED_DIM = 1152
_NUM_CORES = 2
_NUM_SUBCORES = 16
_NUM_WORKERS = _NUM_CORES * _NUM_SUBCORES
_CHUNK = 40


def _gather_body(idx_hbm, table_hbm, out_hbm, idx_v, rows_v, sem):
    wid = lax.axis_index("s") * _NUM_CORES + lax.axis_index("c")
    per_w = idx_hbm.shape[0] // _NUM_WORKERS
    nchunk = per_w // _CHUNK
    base = wid * per_w

    def body(g, carry):
        off = base + g * _CHUNK
        pltpu.sync_copy(idx_hbm.at[pl.ds(off, _CHUNK)], idx_v)
        pltpu.async_copy(table_hbm.at[idx_v], rows_v, sem).wait()
        pltpu.sync_copy(rows_v, out_hbm.at[pl.ds(off, _CHUNK)])
        return carry

    lax.fori_loop(0, nchunk, body, 0)


@jax.jit
def kernel(input, weight):
    b, t = input.shape
    rows = b * t
    idx = input.reshape(rows).astype(jnp.int32)
    mesh = plsc.VectorSubcoreMesh(core_axis_name="c", subcore_axis_name="s")
    out = pl.kernel(
        _gather_body,
        out_type=jax.ShapeDtypeStruct((rows, _EMBED_DIM), jnp.float32),
        mesh=mesh,
        scratch_types=[
            pltpu.VMEM((_CHUNK,), jnp.int32),
            pltpu.VMEM((_CHUNK, _EMBED_DIM), jnp.float32),
            pltpu.SemaphoreType.DMA,
        ],
    )(idx, weight)
    return out.reshape(b, t, _EMBED_DIM)


# speedup vs baseline: 1.3289x; 1.0813x over previous
import jax
import jax.numpy as jnp
from jax import lax
from jax.experimental import pallas as pl
from jax.experimental.pallas import tpu as pltpu
from jax.experimental.pallas import tpu_sc as plsc

_EMBED_DIM = 1152
_NUM_CORES = 2
_NUM_SUBCORES = 16
_NUM_WORKERS = _NUM_CORES * _NUM_SUBCORES
_CHUNK = 40


def _gather_body(idx_hbm, table_hbm, out_hbm,
                 idx_v, rows0, rows1, gsem0, gsem1, ssem0, ssem1):
    wid = lax.axis_index("s") * _NUM_CORES + lax.axis_index("c")
    nchunk = idx_hbm.shape[1]
    per_w = nchunk * _CHUNK
    base = wid * per_w
    bufs = (rows0, rows1)
    gsems = (gsem0, gsem1)
    ssems = (ssem0, ssem1)

    pltpu.sync_copy(idx_hbm.at[wid], idx_v)
    pltpu.async_copy(table_hbm.at[idx_v.at[0]], rows0, gsem0)

    def body(i, carry):
        for b in range(2):
            g = 2 * i + b
            pltpu.make_async_copy(
                table_hbm.at[idx_v.at[g]], bufs[b], gsems[b]).wait()
            pltpu.async_copy(
                bufs[b], out_hbm.at[pl.ds(base + g * _CHUNK, _CHUNK)],
                ssems[b])
            nb = 1 - b
            if b == 0:
                @pl.when(i > 0)
                def _():
                    pltpu.make_async_copy(
                        bufs[nb],
                        out_hbm.at[pl.ds(base + (g - 1) * _CHUNK, _CHUNK)],
                        ssems[nb]).wait()
                pltpu.async_copy(
                    table_hbm.at[idx_v.at[g + 1]], bufs[nb], gsems[nb])
            else:
                pltpu.make_async_copy(
                    bufs[nb],
                    out_hbm.at[pl.ds(base + (g - 1) * _CHUNK, _CHUNK)],
                    ssems[nb]).wait()

                @pl.when(g + 1 < nchunk)
                def _():
                    pltpu.async_copy(
                        table_hbm.at[idx_v.at[g + 1]], bufs[nb], gsems[nb])
        return carry

    lax.fori_loop(0, nchunk // 2, body, 0)
    pltpu.make_async_copy(
        bufs[1], out_hbm.at[pl.ds(base + (nchunk - 1) * _CHUNK, _CHUNK)],
        ssems[1]).wait()


@jax.jit
def kernel(input, weight):
    b, t = input.shape
    rows = b * t
    per_w = rows // _NUM_WORKERS
    nchunk = per_w // _CHUNK
    idx = input.reshape(_NUM_WORKERS, nchunk, _CHUNK).astype(jnp.int32)
    mesh = plsc.VectorSubcoreMesh(core_axis_name="c", subcore_axis_name="s")
    out = pl.kernel(
        _gather_body,
        out_type=jax.ShapeDtypeStruct((rows, _EMBED_DIM), jnp.float32),
        mesh=mesh,
        scratch_types=[
            pltpu.VMEM((nchunk, _CHUNK), jnp.int32),
            pltpu.VMEM((_CHUNK, _EMBED_DIM), jnp.float32),
            pltpu.VMEM((_CHUNK, _EMBED_DIM), jnp.float32),
            pltpu.SemaphoreType.DMA,
            pltpu.SemaphoreType.DMA,
            pltpu.SemaphoreType.DMA,
            pltpu.SemaphoreType.DMA,
        ],
    )(idx, weight)
    return out.reshape(b, t, _EMBED_DIM)
